# baseline (speedup 1.0000x reference)
import functools

import jax
import jax.numpy as jnp
from jax import lax
from jax.experimental import pallas as pl
from jax.experimental.pallas import tpu as pltpu
from jax.experimental.pallas import tpu_sc as plsc

_LAMBDA_2 = 0.01

_B = 16384
_D = 64
_V = 100000
_W = 128
_L = 16
_NC = 2
_NS = 16
_NW = _NC * _NS
_BPW = _B // _NW
_CHUNK = 128
_NCH = _BPW // _CHUNK
_NBUF = 2

_mesh = plsc.VectorSubcoreMesh(core_axis_name="c", subcore_axis_name="s")


@functools.partial(
    pl.kernel,
    mesh=_mesh,
    out_type=[
        jax.ShapeDtypeStruct((_NW, _W), jnp.float32),
        jax.ShapeDtypeStruct((_NW, _W), jnp.float32),
    ],
    scratch_types=[
        pltpu.VMEM((_BPW,), jnp.int32),
        pltpu.VMEM((_BPW,), jnp.int32),
        pltpu.VMEM((_NCH, _CHUNK), jnp.int32),
        pltpu.VMEM((_NCH, _CHUNK), jnp.int32),
        pltpu.VMEM((_BPW,), jnp.float32),
        pltpu.VMEM((_NBUF, _CHUNK, _W), jnp.float32),
        pltpu.VMEM((_NBUF, _CHUNK, _W), jnp.float32),
        pltpu.VMEM((_W,), jnp.float32),
        pltpu.VMEM((_W,), jnp.float32),
        [pltpu.SemaphoreType.DMA] * _NBUF,
    ],
)
def _phi_partials(w_hbm, coo_hbm, idx1_hbm, idx2_hbm, out_sq_hbm, out_abs_hbm,
                  idx1_v, idx2_v, pidx1_v, pidx2_v, coo_v, rows1_v, rows2_v,
                  sq_v, abs_v, sems):
    wid = lax.axis_index("s") * _NC + lax.axis_index("c")
    base = wid * _BPW

    pltpu.sync_copy(idx1_hbm.at[pl.ds(base, _BPW)], idx1_v)
    pltpu.sync_copy(idx2_hbm.at[pl.ds(base, _BPW)], idx2_v)
    pltpu.sync_copy(coo_hbm.at[pl.ds(base, _BPW)], coo_v)

    for j in range(_NCH):
        for g in range(_CHUNK // _L):
            v1 = idx1_v[pl.ds(j * _CHUNK + g * _L, _L)]
            v2 = idx2_v[pl.ds(j * _CHUNK + g * _L, _L)]
            pidx1_v[j, pl.ds(g * _L, _L)] = lax.shift_right_logical(v1, 1)
            pidx2_v[j, pl.ds(g * _L, _L)] = lax.shift_right_logical(v2, 1)

    def fire(j):
        buf = j % _NBUF
        return (
            pltpu.async_copy(w_hbm.at[pidx1_v.at[j]], rows1_v.at[buf], sems[buf]),
            pltpu.async_copy(w_hbm.at[pidx2_v.at[j]], rows2_v.at[buf], sems[buf]),
        )

    inflight = {j: fire(j) for j in range(min(_NBUF, _NCH))}

    zero = jnp.zeros((_L,), jnp.float32)
    acc_sq, acc_abs = zero, zero
    for j in range(_NCH):
        buf = j % _NBUF
        for cp in inflight.pop(j):
            cp.wait()

        def body(g, carry, j=j, buf=buf):
            a_sq, a_abs = carry
            cvec = coo_v[pl.ds(j * _CHUNK + g * _L, _L)]
            hvec1 = lax.shift_left(
                jnp.bitwise_and(idx1_v[pl.ds(j * _CHUNK + g * _L, _L)], 1), 6)
            hvec2 = lax.shift_left(
                jnp.bitwise_and(idx2_v[pl.ds(j * _CHUNK + g * _L, _L)], 1), 6)
            for l in range(_L):
                cb = jnp.full((_L,), cvec[l], dtype=jnp.float32)
                o1 = hvec1[l]
                o2 = hvec2[l]
                r = g * _L + l
                for k in range(_D // _L):
                    a = rows1_v[buf, r, pl.ds(o1 + k * _L, _L)]
                    b = rows2_v[buf, r, pl.ds(o2 + k * _L, _L)]
                    d = cb - a * b
                    a_sq = a_sq + d * d
                    a_abs = a_abs + jnp.abs(a) + jnp.abs(b)
            return a_sq, a_abs

        acc_sq, acc_abs = lax.fori_loop(0, _CHUNK // _L, body, (acc_sq, acc_abs))
        if j + _NBUF < _NCH:
            inflight[j + _NBUF] = fire(j + _NBUF)

    for t in range(_W // _L):
        sq_v[pl.ds(t * _L, _L)] = acc_sq if t == 0 else zero
        abs_v[pl.ds(t * _L, _L)] = acc_abs if t == 0 else zero
    pltpu.sync_copy(sq_v, out_sq_hbm.at[wid])
    pltpu.sync_copy(abs_v, out_abs_hbm.at[wid])


def kernel(w, cooccur, feature_idx1, feature_idx2):
    wp = w.reshape(_V // 2, _W)
    idx1 = feature_idx1.astype(jnp.int32)
    idx2 = feature_idx2.astype(jnp.int32)
    coo = cooccur.reshape(_B)
    sq, ab = _phi_partials(wp, coo, idx1, idx2)
    return jnp.sqrt(jnp.sum(sq)) + (_LAMBDA_2 / 2.0) * jnp.sum(ab)


# speedup vs baseline: 1.0937x; 1.0937x over previous
import functools

import jax
import jax.numpy as jnp
from jax import lax
from jax.experimental import pallas as pl
from jax.experimental.pallas import tpu as pltpu
from jax.experimental.pallas import tpu_sc as plsc

_LAMBDA_2 = 0.01

_B = 16384
_D = 64
_V = 100000
_W = 128
_L = 16
_NC = 2
_NS = 16
_NW = _NC * _NS
_BPW = _B // _NW
_CHUNK = 128
_NCH = _BPW // _CHUNK
_NBUF = 2

_mesh = plsc.VectorSubcoreMesh(core_axis_name="c", subcore_axis_name="s")


@functools.partial(
    pl.kernel,
    mesh=_mesh,
    out_type=[
        jax.ShapeDtypeStruct((_NW, _W), jnp.float32),
        jax.ShapeDtypeStruct((_NW, _W), jnp.float32),
    ],
    scratch_types=[
        pltpu.VMEM((_BPW,), jnp.int32),
        pltpu.VMEM((_BPW,), jnp.int32),
        pltpu.VMEM((_BPW,), jnp.float32),
        pltpu.VMEM((_NBUF, _CHUNK, _W), jnp.float32),
        pltpu.VMEM((_NBUF, _CHUNK, _W), jnp.float32),
        pltpu.VMEM((_W,), jnp.float32),
        pltpu.VMEM((_W,), jnp.float32),
        [pltpu.SemaphoreType.DMA] * _NBUF,
    ],
)
def _phi_partials(w_hbm, coo_hbm, idx1_hbm, idx2_hbm, out_sq_hbm, out_abs_hbm,
                  idx1_v, idx2_v, coo_v, rows1_v, rows2_v, sq_v, abs_v, sems):
    wid = lax.axis_index("s") * _NC + lax.axis_index("c")
    base = wid * _BPW

    pltpu.sync_copy(idx1_hbm.at[pl.ds(base, _BPW)], idx1_v)
    pltpu.sync_copy(idx2_hbm.at[pl.ds(base, _BPW)], idx2_v)
    pltpu.sync_copy(coo_hbm.at[pl.ds(base, _BPW)], coo_v)

    def fire(j):
        buf = j % _NBUF
        return (
            pltpu.async_copy(w_hbm.at[idx1_v.at[pl.ds(j * _CHUNK, _CHUNK)]],
                             rows1_v.at[buf], sems[buf]),
            pltpu.async_copy(w_hbm.at[idx2_v.at[pl.ds(j * _CHUNK, _CHUNK)]],
                             rows2_v.at[buf], sems[buf]),
        )

    inflight = {j: fire(j) for j in range(min(_NBUF, _NCH))}

    zero = jnp.zeros((_L,), jnp.float32)
    acc_sq, acc_abs = zero, zero
    for j in range(_NCH):
        buf = j % _NBUF
        for cp in inflight.pop(j):
            cp.wait()

        def body(g, carry, j=j, buf=buf):
            a_sq, a_abs = carry
            cvec = coo_v[pl.ds(j * _CHUNK + g * _L, _L)]
            for l in range(_L):
                cb = jnp.full((_L,), cvec[l], dtype=jnp.float32)
                r = g * _L + l
                for k in range(_D // _L):
                    a = rows1_v[buf, r, pl.ds(k * _L, _L)]
                    b = rows2_v[buf, r, pl.ds(k * _L, _L)]
                    d = cb - a * b
                    a_sq = a_sq + d * d
                    a_abs = a_abs + jnp.abs(a) + jnp.abs(b)
            return a_sq, a_abs

        acc_sq, acc_abs = lax.fori_loop(0, _CHUNK // _L, body, (acc_sq, acc_abs))
        if j + _NBUF < _NCH:
            inflight[j + _NBUF] = fire(j + _NBUF)

    for t in range(_W // _L):
        sq_v[pl.ds(t * _L, _L)] = acc_sq if t == 0 else zero
        abs_v[pl.ds(t * _L, _L)] = acc_abs if t == 0 else zero
    pltpu.sync_copy(sq_v, out_sq_hbm.at[wid])
    pltpu.sync_copy(abs_v, out_abs_hbm.at[wid])


def kernel(w, cooccur, feature_idx1, feature_idx2):
    wp = jnp.pad(w, ((0, 0), (0, _W - _D)))
    idx1 = feature_idx1.astype(jnp.int32)
    idx2 = feature_idx2.astype(jnp.int32)
    coo = cooccur.reshape(_B)
    sq, ab = _phi_partials(wp, coo, idx1, idx2)
    return jnp.sqrt(jnp.sum(sq)) + (_LAMBDA_2 / 2.0) * jnp.sum(ab)


# speedup vs baseline: 1.7333x; 1.5848x over previous
import functools

import jax
import jax.numpy as jnp
from jax import lax
from jax.experimental import pallas as pl
from jax.experimental.pallas import tpu as pltpu
from jax.experimental.pallas import tpu_sc as plsc

_LAMBDA_2 = 0.01

_B = 16384
_D = 64
_V = 100000
_L = 16
_NC = 2
_NS = 16
_NW = _NC * _NS
_DPW = _D // _NW
_NH = 2
_BH = _B // _NH
_OUTW = 128

_mesh = plsc.VectorSubcoreMesh(core_axis_name="c", subcore_axis_name="s")


@functools.partial(
    pl.kernel,
    mesh=_mesh,
    compiler_params=pltpu.CompilerParams(needs_layout_passes=False),
    out_type=[
        jax.ShapeDtypeStruct((_NW, _OUTW), jnp.float32),
        jax.ShapeDtypeStruct((_NW, _OUTW), jnp.float32),
    ],
    scratch_types=[
        pltpu.VMEM((_V,), jnp.float32),
        pltpu.VMEM((_BH,), jnp.int32),
        pltpu.VMEM((_BH,), jnp.int32),
        pltpu.VMEM((_BH,), jnp.float32),
        pltpu.VMEM((_OUTW,), jnp.float32),
        pltpu.VMEM((_OUTW,), jnp.float32),
        pltpu.SemaphoreType.DMA,
    ],
)
def _phi_partials(wt_hbm, coo_hbm, idx1_hbm, idx2_hbm, out_sq_hbm, out_abs_hbm,
                  row_v, idx1_v, idx2_v, coo_v, sq_v, abs_v, sem):
    wid = lax.axis_index("s") * _NC + lax.axis_index("c")

    zero = jnp.zeros((_L,), jnp.float32)
    acc_sq, acc_abs = zero, zero
    for p in range(_DPW):
        d = wid * _DPW + p
        row_cp = pltpu.async_copy(wt_hbm.at[d], row_v, sem)
        for h in range(_NH):
            pltpu.sync_copy(idx1_hbm.at[pl.ds(h * _BH, _BH)], idx1_v)
            pltpu.sync_copy(idx2_hbm.at[pl.ds(h * _BH, _BH)], idx2_v)
            pltpu.sync_copy(coo_hbm.at[pl.ds(h * _BH, _BH)], coo_v)
            if h == 0:
                row_cp.wait()

            def body(g, carry):
                a_sq, a_abs = carry
                i1 = idx1_v[pl.ds(g * _L, _L)]
                i2 = idx2_v[pl.ds(g * _L, _L)]
                cvec = coo_v[pl.ds(g * _L, _L)]
                g1 = plsc.load_gather(row_v, [i1])
                g2 = plsc.load_gather(row_v, [i2])
                dd = cvec - g1 * g2
                a_sq = a_sq + dd * dd
                a_abs = a_abs + jnp.abs(g1) + jnp.abs(g2)
                return a_sq, a_abs

            acc_sq, acc_abs = lax.fori_loop(0, _BH // _L, body,
                                            (acc_sq, acc_abs))

    for t in range(_OUTW // _L):
        sq_v[pl.ds(t * _L, _L)] = acc_sq if t == 0 else zero
        abs_v[pl.ds(t * _L, _L)] = acc_abs if t == 0 else zero
    pltpu.sync_copy(sq_v, out_sq_hbm.at[wid])
    pltpu.sync_copy(abs_v, out_abs_hbm.at[wid])


def kernel(w, cooccur, feature_idx1, feature_idx2):
    wt = w.T
    idx1 = feature_idx1.astype(jnp.int32)
    idx2 = feature_idx2.astype(jnp.int32)
    coo = cooccur.reshape(_B)
    sq, ab = _phi_partials(wt, coo, idx1, idx2)
    return jnp.sqrt(jnp.sum(sq)) + (_LAMBDA_2 / 2.0) * jnp.sum(ab)
